# baseline (device time: 87334 ns/iter reference)
import jax
import jax.numpy as jnp
from jax import lax
from jax.experimental import pallas as pl
from jax.experimental.pallas import tpu as pltpu

N_DEV = 32

L_OFFS = [1, 31, 8, 24, 16, 7, 25, 4, 28, 9, 23, 3, 29, 2, 30, 5, 15,
          17, 27, 6, 26, 12, 20, 11, 21, 13, 19, 10, 22, 14, 18]
OFFS = [0] + L_OFFS


def kernel(x, w_mat):
    m_total, k_local = x.shape
    k_total, n_out = w_mat.shape
    m_blk = m_total // N_DEV
    assert m_blk == k_local and k_total == m_total

    x = x.astype(jnp.bfloat16)

    def body(x_ref, w_ref, out_ref, buf_ref, wb0_ref, wb1_ref,
             send_sems, recv_sems, wcopy_sems):
        my = lax.axis_index("i")
        wb_refs = [wb0_ref, wb1_ref]

        def blk(g):
            return (my - OFFS[g]) % N_DEV

        def w_copy(g):
            return pltpu.make_async_copy(
                w_ref.at[pl.ds(blk(g) * m_blk, m_blk), :],
                wb_refs[g % 2].at[...],
                wcopy_sems.at[g % 2],
            )

        w_copy(0).start()
        w_copy(1).start()

        barrier_sem = pltpu.get_barrier_semaphore()
        for d in range(1, N_DEV):
            pl.semaphore_signal(
                barrier_sem, inc=1,
                device_id=((my + d) % N_DEV,),
                device_id_type=pl.DeviceIdType.MESH,
            )
        pl.semaphore_wait(barrier_sem, N_DEV - 1)

        sends = []
        for d in L_OFFS:
            tgt = (my + d) % N_DEV
            rdma = pltpu.make_async_remote_copy(
                src_ref=x_ref.at[pl.ds(tgt * m_blk, m_blk), :],
                dst_ref=buf_ref.at[pl.ds(my * m_blk, m_blk), :],
                send_sem=send_sems.at[d - 1],
                recv_sem=recv_sems.at[d - 1],
                device_id=(tgt,),
                device_id_type=pl.DeviceIdType.MESH,
            )
            rdma.start()
            sends.append(rdma)

        for g in range(N_DEV):
            if g + 2 < N_DEV:
                w_copy(g + 2).start()
            w_copy(g).wait()
            if g == 0:
                a = x_ref[pl.ds(my * m_blk, m_blk), :]
            else:
                recv = pltpu.make_async_remote_copy(
                    src_ref=x_ref.at[pl.ds(0, m_blk), :],
                    dst_ref=buf_ref.at[pl.ds(blk(g) * m_blk, m_blk), :],
                    send_sem=send_sems.at[OFFS[g] - 1],
                    recv_sem=recv_sems.at[OFFS[g] - 1],
                    device_id=(my,),
                    device_id_type=pl.DeviceIdType.MESH,
                )
                recv.wait_recv()
                a = buf_ref[pl.ds(blk(g) * m_blk, m_blk), :]
            part = jnp.dot(a.astype(jnp.float32), wb_refs[g % 2][...],
                           preferred_element_type=jnp.float32)
            if g == 0:
                out_ref[...] = part
            else:
                out_ref[...] += part

        for rdma in sends:
            rdma.wait_send()

    return pl.pallas_call(
        body,
        out_shape=jax.ShapeDtypeStruct((m_blk, n_out), jnp.float32),
        in_specs=[
            pl.BlockSpec(memory_space=pltpu.VMEM),
            pl.BlockSpec(memory_space=pltpu.MemorySpace.HBM),
        ],
        out_specs=pl.BlockSpec(memory_space=pltpu.VMEM),
        scratch_shapes=[
            pltpu.VMEM((m_total, k_local), jnp.bfloat16),
            pltpu.VMEM((m_blk, n_out), jnp.float32),
            pltpu.VMEM((m_blk, n_out), jnp.float32),
            pltpu.SemaphoreType.DMA((N_DEV - 1,)),
            pltpu.SemaphoreType.DMA((N_DEV - 1,)),
            pltpu.SemaphoreType.DMA((2,)),
        ],
        compiler_params=pltpu.CompilerParams(
            collective_id=0,
            vmem_limit_bytes=100 * 1024 * 1024,
        ),
    )(x, w_mat)


# device time: 83914 ns/iter; 1.0408x vs baseline; 1.0408x over previous
import jax
import jax.numpy as jnp
from jax import lax
from jax.experimental import pallas as pl
from jax.experimental.pallas import tpu as pltpu

N_DEV = 32

L_OFFS = [1, 31, 8, 24, 16, 7, 25, 4, 28, 9, 23, 3, 29, 2, 30, 5, 15,
          17, 27, 6, 26, 12, 20, 11, 21, 13, 19, 10, 22, 14, 18]
OFFS = [0] + L_OFFS


def kernel(x, w_mat):
    m_total, k_local = x.shape
    k_total, n_out = w_mat.shape
    m_blk = m_total // N_DEV
    assert m_blk == k_local and k_total == m_total

    def body(x_ref, w_ref, out_ref, xs_ref, xb_ref, buf_ref, wb0_ref, wb1_ref,
             send_sems, recv_sems, wcopy_sems, xcopy_sem):
        my = lax.axis_index("i")
        wb_refs = [wb0_ref, wb1_ref]

        def blk(g):
            return (my - OFFS[g]) % N_DEV

        def w_copy(g):
            return pltpu.make_async_copy(
                w_ref.at[pl.ds(blk(g) * m_blk, m_blk), :],
                wb_refs[g % 2].at[...],
                wcopy_sems.at[g % 2],
            )

        w_copy(0).start()
        w_copy(1).start()
        xcopy = pltpu.make_async_copy(x_ref, xs_ref, xcopy_sem)
        xcopy.start()
        xcopy.wait()
        xb_ref[...] = xs_ref[...].astype(jnp.bfloat16)

        barrier_sem = pltpu.get_barrier_semaphore()
        for d in range(1, N_DEV):
            pl.semaphore_signal(
                barrier_sem, inc=1,
                device_id=((my + d) % N_DEV,),
                device_id_type=pl.DeviceIdType.MESH,
            )
        pl.semaphore_wait(barrier_sem, N_DEV - 1)

        sends = []
        for d in L_OFFS:
            tgt = (my + d) % N_DEV
            rdma = pltpu.make_async_remote_copy(
                src_ref=xb_ref.at[pl.ds(tgt * m_blk, m_blk), :],
                dst_ref=buf_ref.at[pl.ds(my * m_blk, m_blk), :],
                send_sem=send_sems.at[d - 1],
                recv_sem=recv_sems.at[d - 1],
                device_id=(tgt,),
                device_id_type=pl.DeviceIdType.MESH,
            )
            rdma.start()
            sends.append(rdma)

        for g in range(N_DEV):
            if g + 2 < N_DEV:
                w_copy(g + 2).start()
            w_copy(g).wait()
            if g == 0:
                a = xb_ref[pl.ds(my * m_blk, m_blk), :]
            else:
                recv = pltpu.make_async_remote_copy(
                    src_ref=xb_ref.at[pl.ds(0, m_blk), :],
                    dst_ref=buf_ref.at[pl.ds(blk(g) * m_blk, m_blk), :],
                    send_sem=send_sems.at[OFFS[g] - 1],
                    recv_sem=recv_sems.at[OFFS[g] - 1],
                    device_id=(my,),
                    device_id_type=pl.DeviceIdType.MESH,
                )
                recv.wait_recv()
                a = buf_ref[pl.ds(blk(g) * m_blk, m_blk), :]
            part = jnp.dot(a.astype(jnp.float32), wb_refs[g % 2][...],
                           preferred_element_type=jnp.float32)
            if g == 0:
                out_ref[...] = part
            else:
                out_ref[...] += part

        for rdma in sends:
            rdma.wait_send()

    return pl.pallas_call(
        body,
        out_shape=jax.ShapeDtypeStruct((m_blk, n_out), jnp.float32),
        in_specs=[
            pl.BlockSpec(memory_space=pltpu.MemorySpace.HBM),
            pl.BlockSpec(memory_space=pltpu.MemorySpace.HBM),
        ],
        out_specs=pl.BlockSpec(memory_space=pltpu.VMEM),
        scratch_shapes=[
            pltpu.VMEM((m_total, k_local), jnp.float32),
            pltpu.VMEM((m_total, k_local), jnp.bfloat16),
            pltpu.VMEM((m_total, k_local), jnp.bfloat16),
            pltpu.VMEM((m_blk, n_out), jnp.float32),
            pltpu.VMEM((m_blk, n_out), jnp.float32),
            pltpu.SemaphoreType.DMA((N_DEV - 1,)),
            pltpu.SemaphoreType.DMA((N_DEV - 1,)),
            pltpu.SemaphoreType.DMA((2,)),
            pltpu.SemaphoreType.DMA,
        ],
        compiler_params=pltpu.CompilerParams(
            collective_id=0,
            vmem_limit_bytes=100 * 1024 * 1024,
        ),
    )(x, w_mat)


# device time: 83649 ns/iter; 1.0441x vs baseline; 1.0032x over previous
import jax
import jax.numpy as jnp
from jax import lax
from jax.experimental import pallas as pl
from jax.experimental.pallas import tpu as pltpu

N_DEV = 32

L_OFFS = [1, 31, 8, 24, 16, 7, 25, 4, 28, 9, 23, 3, 29, 2, 30, 5, 15,
          17, 27, 6, 26, 12, 20, 11, 21, 13, 19, 10, 22, 14, 18]
OFFS = [0] + L_OFFS


def kernel(x, w_mat):
    m_total, k_local = x.shape
    k_total, n_out = w_mat.shape
    m_blk = m_total // N_DEV
    assert m_blk == k_local and k_total == m_total

    def body(x_ref, w_ref, out_ref, xs_ref, xb_ref, buf_ref, wb0_ref, wb1_ref,
             send_sems, recv_sems, wcopy_sems, xcopy_sem):
        my = lax.axis_index("i")
        wb_refs = [wb0_ref, wb1_ref]

        def blk(g):
            return (my - OFFS[g]) % N_DEV

        def w_copy(g):
            return pltpu.make_async_copy(
                w_ref.at[pl.ds(blk(g) * m_blk, m_blk), :],
                wb_refs[g % 2].at[...],
                wcopy_sems.at[g % 2],
            )

        w_copy(0).start()
        w_copy(1).start()
        xcopy = pltpu.make_async_copy(x_ref, xs_ref, xcopy_sem)
        xcopy.start()
        xcopy.wait()
        xb_ref[...] = xs_ref[...].astype(jnp.bfloat16)

        barrier_sem = pltpu.get_barrier_semaphore()
        for d in range(1, N_DEV):
            pl.semaphore_signal(
                barrier_sem, inc=1,
                device_id=((my + d) % N_DEV,),
                device_id_type=pl.DeviceIdType.MESH,
            )
        pl.semaphore_wait(barrier_sem, N_DEV - 1)

        sends = []
        for d in L_OFFS:
            tgt = (my + d) % N_DEV
            rdma = pltpu.make_async_remote_copy(
                src_ref=xb_ref.at[pl.ds(tgt * m_blk, m_blk), :],
                dst_ref=buf_ref.at[pl.ds(my * m_blk, m_blk), :],
                send_sem=send_sems.at[d - 1],
                recv_sem=recv_sems.at[d - 1],
                device_id=(tgt,),
                device_id_type=pl.DeviceIdType.MESH,
            )
            rdma.start()
            sends.append(rdma)

        for g in range(N_DEV):
            w_copy(g).wait()
            if g + 2 < N_DEV:
                w_copy(g + 2).start()
            if g == 0:
                a = xb_ref[pl.ds(my * m_blk, m_blk), :]
            else:
                recv = pltpu.make_async_remote_copy(
                    src_ref=xb_ref.at[pl.ds(0, m_blk), :],
                    dst_ref=buf_ref.at[pl.ds(blk(g) * m_blk, m_blk), :],
                    send_sem=send_sems.at[OFFS[g] - 1],
                    recv_sem=recv_sems.at[OFFS[g] - 1],
                    device_id=(my,),
                    device_id_type=pl.DeviceIdType.MESH,
                )
                recv.wait_recv()
                a = buf_ref[pl.ds(blk(g) * m_blk, m_blk), :]
            part = jnp.dot(a.astype(jnp.float32), wb_refs[g % 2][...],
                           preferred_element_type=jnp.float32)
            if g == 0:
                out_ref[...] = part
            else:
                out_ref[...] += part

        for rdma in sends:
            rdma.wait_send()

    return pl.pallas_call(
        body,
        out_shape=jax.ShapeDtypeStruct((m_blk, n_out), jnp.float32),
        in_specs=[
            pl.BlockSpec(memory_space=pltpu.MemorySpace.HBM),
            pl.BlockSpec(memory_space=pltpu.MemorySpace.HBM),
        ],
        out_specs=pl.BlockSpec(memory_space=pltpu.VMEM),
        scratch_shapes=[
            pltpu.VMEM((m_total, k_local), jnp.float32),
            pltpu.VMEM((m_total, k_local), jnp.bfloat16),
            pltpu.VMEM((m_total, k_local), jnp.bfloat16),
            pltpu.VMEM((m_blk, n_out), jnp.float32),
            pltpu.VMEM((m_blk, n_out), jnp.float32),
            pltpu.SemaphoreType.DMA((N_DEV - 1,)),
            pltpu.SemaphoreType.DMA((N_DEV - 1,)),
            pltpu.SemaphoreType.DMA((2,)),
            pltpu.SemaphoreType.DMA,
        ],
        compiler_params=pltpu.CompilerParams(
            collective_id=0,
            vmem_limit_bytes=100 * 1024 * 1024,
        ),
    )(x, w_mat)
